# baseline (speedup 1.0000x reference)
import jax
import jax.numpy as jnp
from jax import lax
from jax.experimental import pallas as pl
from jax.experimental.pallas import tpu as pltpu
from jax.experimental.pallas import tpu_sc as plsc

NUM_RELS = 38
N_NODES = 100000
N_EDGES = 6400000

NC = 2
NS = 16
L = 16
NW = NC * NS

EPW = N_EDGES // NW
CHUNK = 4000
NCHUNK = EPW // CHUNK
BINS = NUM_RELS * 9
BINS_PAD = 352
SW_PAD = 512
UNROLL = 10


def _body(src_hbm, dst_hbm, et_hbm, nt_hbm, sw_hbm, out_hbm,
          nt_v, sw_v, src_v0, dst_v0, et_v0, src_v1, dst_v1, et_v1,
          acc_v, sem0, sem1):
  bufs = ((src_v0, dst_v0, et_v0), (src_v1, dst_v1, et_v1))
  wid = lax.axis_index("s") * NC + lax.axis_index("c")
  base0 = pl.multiple_of(wid * EPW, 8)

  pltpu.sync_copy(nt_hbm, nt_v)
  pltpu.sync_copy(sw_hbm, sw_v)

  def issue(c, slot, sem):
    base = pl.multiple_of(base0 + c * CHUNK, 8)
    sb, db, tb = bufs[slot]
    pltpu.make_async_copy(src_hbm.at[pl.ds(base, CHUNK)], sb, sem).start()
    pltpu.make_async_copy(dst_hbm.at[pl.ds(base, CHUNK)], db, sem).start()
    pltpu.make_async_copy(et_hbm.at[pl.ds(base, CHUNK)], tb, sem).start()

  def drain(slot, sem):
    sb, db, tb = bufs[slot]
    pltpu.make_async_copy(src_hbm.at[pl.ds(base0, CHUNK)], sb, sem).wait()
    pltpu.make_async_copy(dst_hbm.at[pl.ds(base0, CHUNK)], db, sem).wait()
    pltpu.make_async_copy(et_hbm.at[pl.ds(base0, CHUNK)], tb, sem).wait()

  issue(0, 0, sem0)
  issue(1, 1, sem1)

  def compute(slot, acc):
    sb, db, tb = bufs[slot]
    def it(i, acc):
      off = pl.multiple_of(i * (L * UNROLL), L)
      accs = list(acc)
      for u in range(UNROLL):
        o = off + u * L
        s = sb[pl.ds(o, L)]
        d = db[pl.ds(o, L)]
        t = tb[pl.ds(o, L)]
        ns = plsc.load_gather(nt_v, [s])
        nd = plsc.load_gather(nt_v, [d])
        enc = t * 9 + ns * 3 + nd
        accs[u] = accs[u] + plsc.load_gather(sw_v, [enc])
      return tuple(accs)
    return lax.fori_loop(0, CHUNK // (L * UNROLL), it, acc)

  def gloop(g, acc):
    last = g == (NCHUNK // 2 - 1)
    drain(0, sem0)
    acc = compute(0, acc)
    @pl.when(jnp.logical_not(last))
    def _i0():
      issue(2 * g + 2, 0, sem0)
    drain(1, sem1)
    acc = compute(1, acc)
    @pl.when(jnp.logical_not(last))
    def _i1():
      issue(2 * g + 3, 1, sem1)
    return acc

  zero = jnp.zeros((L,), jnp.float32)
  accs = lax.fori_loop(0, NCHUNK // 2, gloop, (zero,) * UNROLL)
  acc = accs[0]
  for u in range(1, UNROLL):
    acc = acc + accs[u]

  acc_v[...] = acc
  pltpu.sync_copy(acc_v, out_hbm.at[wid])


@jax.jit
def _run(src, dst, et, nt, sw_pad):
  mesh = plsc.VectorSubcoreMesh(core_axis_name="c", subcore_axis_name="s")
  partials = pl.kernel(
      _body,
      out_type=jax.ShapeDtypeStruct((NW, L), jnp.float32),
      mesh=mesh,
      compiler_params=pltpu.CompilerParams(needs_layout_passes=False),
      scratch_types=[
          pltpu.VMEM((N_NODES,), jnp.int32),
          pltpu.VMEM((SW_PAD,), jnp.float32),
          pltpu.VMEM((CHUNK,), jnp.int32),
          pltpu.VMEM((CHUNK,), jnp.int32),
          pltpu.VMEM((CHUNK,), jnp.int32),
          pltpu.VMEM((CHUNK,), jnp.int32),
          pltpu.VMEM((CHUNK,), jnp.int32),
          pltpu.VMEM((CHUNK,), jnp.int32),
          pltpu.VMEM((L,), jnp.float32),
          pltpu.SemaphoreType.DMA,
          pltpu.SemaphoreType.DMA,
      ],
  )(src, dst, et, nt, sw_pad)
  return partials.sum()


def kernel(node_type, edge_type, edge_index, text, scorer_weight):
  src = edge_index[0]
  dst = edge_index[1]
  sw = jnp.zeros((SW_PAD,), jnp.float32).at[:BINS].set(
      scorer_weight.reshape(-1))
  return _run(src, dst, edge_type, node_type, sw)


# speedup vs baseline: 1.1298x; 1.1298x over previous
import jax
import jax.numpy as jnp
from jax import lax
from jax.experimental import pallas as pl
from jax.experimental.pallas import tpu as pltpu
from jax.experimental.pallas import tpu_sc as plsc

NUM_RELS = 38
N_NODES = 100000
N_EDGES = 6400000

NC = 2
NS = 16
L = 16
NW = NC * NS

EPW = N_EDGES // NW
CHUNK = 4000
NCHUNK = EPW // CHUNK
BINS = NUM_RELS * 9
BINS_PAD = 352
SW_PAD = 512
UNROLL = 10


def _body(ei_hbm, et_hbm, nt_hbm, sw_hbm, out_hbm,
          nt_v, sw_v, src_v0, dst_v0, et_v0, src_v1, dst_v1, et_v1,
          acc_v, sem0, sem1):
  bufs = ((src_v0, dst_v0, et_v0), (src_v1, dst_v1, et_v1))
  wid = lax.axis_index("s") * NC + lax.axis_index("c")
  base0 = pl.multiple_of(wid * EPW, 8)

  pltpu.sync_copy(nt_hbm, nt_v)
  pltpu.sync_copy(sw_hbm, sw_v)

  def issue(c, slot, sem):
    base = pl.multiple_of(base0 + c * CHUNK, 8)
    sb, db, tb = bufs[slot]
    pltpu.make_async_copy(ei_hbm.at[pl.ds(base, CHUNK)], sb, sem).start()
    pltpu.make_async_copy(ei_hbm.at[pl.ds(N_EDGES + base, CHUNK)], db, sem).start()
    pltpu.make_async_copy(et_hbm.at[pl.ds(base, CHUNK)], tb, sem).start()

  def drain(slot, sem):
    sb, db, tb = bufs[slot]
    pltpu.make_async_copy(ei_hbm.at[pl.ds(base0, CHUNK)], sb, sem).wait()
    pltpu.make_async_copy(ei_hbm.at[pl.ds(base0, CHUNK)], db, sem).wait()
    pltpu.make_async_copy(et_hbm.at[pl.ds(base0, CHUNK)], tb, sem).wait()

  issue(0, 0, sem0)
  issue(1, 1, sem1)

  def compute(slot, acc):
    sb, db, tb = bufs[slot]
    def it(i, acc):
      off = pl.multiple_of(i * (L * UNROLL), L)
      accs = list(acc)
      for u in range(UNROLL):
        o = off + u * L
        s = sb[pl.ds(o, L)]
        d = db[pl.ds(o, L)]
        t = tb[pl.ds(o, L)]
        ns = plsc.load_gather(nt_v, [s])
        nd = plsc.load_gather(nt_v, [d])
        enc = t * 9 + ns * 3 + nd
        accs[u] = accs[u] + plsc.load_gather(sw_v, [enc])
      return tuple(accs)
    return lax.fori_loop(0, CHUNK // (L * UNROLL), it, acc)

  def gloop(g, acc):
    last = g == (NCHUNK // 2 - 1)
    drain(0, sem0)
    acc = compute(0, acc)
    @pl.when(jnp.logical_not(last))
    def _i0():
      issue(2 * g + 2, 0, sem0)
    drain(1, sem1)
    acc = compute(1, acc)
    @pl.when(jnp.logical_not(last))
    def _i1():
      issue(2 * g + 3, 1, sem1)
    return acc

  zero = jnp.zeros((L,), jnp.float32)
  accs = lax.fori_loop(0, NCHUNK // 2, gloop, (zero,) * UNROLL)
  acc = accs[0]
  for u in range(1, UNROLL):
    acc = acc + accs[u]

  acc_v[...] = acc
  pltpu.sync_copy(acc_v, out_hbm.at[wid])


@jax.jit
def _run(ei, et, nt, sw_pad):
  mesh = plsc.VectorSubcoreMesh(core_axis_name="c", subcore_axis_name="s")
  partials = pl.kernel(
      _body,
      out_type=jax.ShapeDtypeStruct((NW, L), jnp.float32),
      mesh=mesh,
      compiler_params=pltpu.CompilerParams(needs_layout_passes=False),
      scratch_types=[
          pltpu.VMEM((N_NODES,), jnp.int32),
          pltpu.VMEM((SW_PAD,), jnp.float32),
          pltpu.VMEM((CHUNK,), jnp.int32),
          pltpu.VMEM((CHUNK,), jnp.int32),
          pltpu.VMEM((CHUNK,), jnp.int32),
          pltpu.VMEM((CHUNK,), jnp.int32),
          pltpu.VMEM((CHUNK,), jnp.int32),
          pltpu.VMEM((CHUNK,), jnp.int32),
          pltpu.VMEM((L,), jnp.float32),
          pltpu.SemaphoreType.DMA,
          pltpu.SemaphoreType.DMA,
      ],
  )(ei, et, nt, sw_pad)
  return partials.sum()


def kernel(node_type, edge_type, edge_index, text, scorer_weight):
  sw = jnp.zeros((SW_PAD,), jnp.float32).at[:BINS].set(
      scorer_weight.reshape(-1))
  return _run(edge_index.reshape(-1), edge_type, node_type, sw)


# speedup vs baseline: 1.4481x; 1.2817x over previous
import jax
import jax.numpy as jnp
from jax import lax
from jax.experimental import pallas as pl
from jax.experimental.pallas import tpu as pltpu
from jax.experimental.pallas import tpu_sc as plsc

NUM_RELS = 38
N_NODES = 100000
N_EDGES = 6400000

NC = 2
NS = 16
L = 16
NW = NC * NS

CHUNK = 2560
NCHUNK_TOT = N_EDGES // CHUNK
NCHUNK_W = NCHUNK_TOT // NW
NEXTRA = NCHUNK_TOT - NCHUNK_W * NW
BINS = NUM_RELS * 9
SW_PAD = 512
UNROLL = 10

assert CHUNK % (L * UNROLL) == 0
assert NCHUNK_W % 2 == 0
assert NCHUNK_W * NW + NEXTRA == NCHUNK_TOT


def _body(ei_hbm, et_hbm, nt_hbm, sw_hbm, out_hbm,
          nt_v, sw_v, ei_v0, et_v0, ei_v1, et_v1, acc_v, sem0, sem1):
  bufs = ((ei_v0, et_v0), (ei_v1, et_v1))
  wid = lax.axis_index("s") * NC + lax.axis_index("c")
  c0 = wid * NCHUNK_W

  pltpu.sync_copy(nt_hbm, nt_v)
  pltpu.sync_copy(sw_hbm, sw_v)

  def issue(c, slot, sem):
    base = pl.multiple_of(c * CHUNK, 128)
    eb, tb = bufs[slot]
    pltpu.make_async_copy(ei_hbm.at[:, pl.ds(base, CHUNK)], eb, sem).start()
    pltpu.make_async_copy(et_hbm.at[pl.ds(base, CHUNK)], tb, sem).start()

  def drain(slot, sem):
    eb, tb = bufs[slot]
    pltpu.make_async_copy(ei_hbm.at[:, pl.ds(0, CHUNK)], eb, sem).wait()
    pltpu.make_async_copy(et_hbm.at[pl.ds(0, CHUNK)], tb, sem).wait()

  issue(c0, 0, sem0)
  issue(c0 + 1, 1, sem1)

  def compute(slot, acc):
    eb, tb = bufs[slot]
    def it(i, acc):
      off = pl.multiple_of(i * (L * UNROLL), L)
      accs = list(acc)
      for u in range(UNROLL):
        o = off + u * L
        s = eb[0, pl.ds(o, L)]
        d = eb[1, pl.ds(o, L)]
        t = tb[pl.ds(o, L)]
        ns = plsc.load_gather(nt_v, [s])
        nd = plsc.load_gather(nt_v, [d])
        enc = t * 9 + ns * 3 + nd
        accs[u] = accs[u] + plsc.load_gather(sw_v, [enc])
      return tuple(accs)
    return lax.fori_loop(0, CHUNK // (L * UNROLL), it, acc)

  def gloop(g, acc):
    last = g == (NCHUNK_W // 2 - 1)
    drain(0, sem0)
    acc = compute(0, acc)
    @pl.when(jnp.logical_not(last))
    def _i0():
      issue(c0 + 2 * g + 2, 0, sem0)
    drain(1, sem1)
    acc = compute(1, acc)
    @pl.when(jnp.logical_not(last))
    def _i1():
      issue(c0 + 2 * g + 3, 1, sem1)
    return acc

  zero = jnp.zeros((L,), jnp.float32)
  acc = lax.fori_loop(0, NCHUNK_W // 2, gloop, (zero,) * UNROLL)

  @pl.when(wid < NEXTRA)
  def _ix():
    issue(NCHUNK_W * NW + wid, 0, sem0)

  def extra(acc):
    drain(0, sem0)
    return compute(0, acc)

  acc = lax.cond(wid < NEXTRA, extra, lambda a: a, acc)

  total = acc[0]
  for u in range(1, UNROLL):
    total = total + acc[u]
  acc_v[...] = total
  pltpu.sync_copy(acc_v, out_hbm.at[wid])


@jax.jit
def _run(ei, et, nt, sw_pad):
  mesh = plsc.VectorSubcoreMesh(core_axis_name="c", subcore_axis_name="s")
  partials = pl.kernel(
      _body,
      out_type=jax.ShapeDtypeStruct((NW, L), jnp.float32),
      mesh=mesh,
      compiler_params=pltpu.CompilerParams(needs_layout_passes=False),
      scratch_types=[
          pltpu.VMEM((N_NODES,), jnp.int32),
          pltpu.VMEM((SW_PAD,), jnp.float32),
          pltpu.VMEM((2, CHUNK), jnp.int32),
          pltpu.VMEM((CHUNK,), jnp.int32),
          pltpu.VMEM((2, CHUNK), jnp.int32),
          pltpu.VMEM((CHUNK,), jnp.int32),
          pltpu.VMEM((L,), jnp.float32),
          pltpu.SemaphoreType.DMA,
          pltpu.SemaphoreType.DMA,
      ],
  )(ei, et, nt, sw_pad)
  return partials.sum()


def kernel(node_type, edge_type, edge_index, text, scorer_weight):
  sw = jnp.zeros((SW_PAD,), jnp.float32).at[:BINS].set(
      scorer_weight.reshape(-1))
  return _run(edge_index, edge_type, node_type, sw)
